# baseline (speedup 1.0000x reference)
import jax
import jax.numpy as jnp
from jax import lax
from jax.experimental import pallas as pl
from jax.experimental.pallas import tpu as pltpu
from jax.experimental.pallas import tpu_sc as plsc

N = 10000
E = 320000
R = 45
IN_CH = 128
HID = 16
RP = 48
NP = 10112
NRP = NP * RP
NRY = N * RP
EP = 327680
ECH = 2048
ROWS = ECH // 128
NSUB = 16
NCORE = 2
BIN_T = NRP // NSUB
BIN_CH = 5056
ACC_T = NP // NSUB
NCH_E = EP // ECH // (NSUB * NCORE)
NCH_C = EP // ECH // NSUB

_f32 = jnp.float32
_i32 = jnp.int32


def _tc1_body(x_ref, w_ref, r_ref, b_ref, y_ref, xr_ref):
    xb = x_ref[...]
    y = jnp.dot(xb, w_ref[...], preferred_element_type=_f32)
    y_ref[...] = y.reshape(y_ref.shape)
    xr_ref[...] = jnp.dot(xb, r_ref[...], preferred_element_type=_f32) + b_ref[...]


def _tc2_body(p0_ref, p1_ref, xr_ref, w_ref, r_ref, b_ref, y_ref, xr2_ref):
    h = jnp.maximum(p0_ref[:, :HID] + p1_ref[:, :HID] + xr_ref[...], 0.0)
    y = jnp.dot(h, w_ref[...], preferred_element_type=_f32)
    y_ref[...] = y.reshape(y_ref.shape)
    xr2_ref[...] = jnp.dot(h, r_ref[...], preferred_element_type=_f32) + b_ref[...]


def _tc3_body(p0_ref, p1_ref, xr_ref, o_ref):
    o_ref[...] = p0_ref[:, :HID] + p1_ref[:, :HID] + xr_ref[...]


def _tc1(x, w1c, root1, b1):
    blk = 2000
    return pl.pallas_call(
        _tc1_body,
        grid=(N // blk,),
        in_specs=[
            pl.BlockSpec((blk, IN_CH), lambda i: (i, 0)),
            pl.BlockSpec((IN_CH, RP * HID), lambda i: (0, 0)),
            pl.BlockSpec((IN_CH, HID), lambda i: (0, 0)),
            pl.BlockSpec((1, HID), lambda i: (0, 0)),
        ],
        out_specs=[
            pl.BlockSpec((blk * 6, 128), lambda i: (i, 0)),
            pl.BlockSpec((blk, HID), lambda i: (i, 0)),
        ],
        out_shape=[
            jax.ShapeDtypeStruct((N * 6, 128), _f32),
            jax.ShapeDtypeStruct((N, HID), _f32),
        ],
    )(x, w1c, root1, b1)


def _tc2(p0, p1, xr1, w2c, root2, b2):
    blk = 2000
    return pl.pallas_call(
        _tc2_body,
        grid=(N // blk,),
        in_specs=[
            pl.BlockSpec((blk, 128), lambda i: (i, 0)),
            pl.BlockSpec((blk, 128), lambda i: (i, 0)),
            pl.BlockSpec((blk, HID), lambda i: (i, 0)),
            pl.BlockSpec((HID, RP * HID), lambda i: (0, 0)),
            pl.BlockSpec((HID, HID), lambda i: (0, 0)),
            pl.BlockSpec((1, HID), lambda i: (0, 0)),
        ],
        out_specs=[
            pl.BlockSpec((blk * 6, 128), lambda i: (i, 0)),
            pl.BlockSpec((blk, HID), lambda i: (i, 0)),
        ],
        out_shape=[
            jax.ShapeDtypeStruct((N * 6, 128), _f32),
            jax.ShapeDtypeStruct((N, HID), _f32),
        ],
    )(p0, p1, xr1, w2c, root2, b2)


def _tc3(p0, p1, xr2):
    blk = 2000
    return pl.pallas_call(
        _tc3_body,
        grid=(N // blk,),
        in_specs=[
            pl.BlockSpec((blk, 128), lambda i: (i, 0)),
            pl.BlockSpec((blk, 128), lambda i: (i, 0)),
            pl.BlockSpec((blk, HID), lambda i: (i, 0)),
        ],
        out_specs=pl.BlockSpec((blk, HID), lambda i: (i, 0)),
        out_shape=jax.ShapeDtypeStruct((N, HID), _f32),
    )(p0, p1, xr2)


def _zero_vec16(ref, nvec):
    @pl.loop(0, nvec)
    def _(i):
        ref[pl.ds(i * 16, 16)] = jnp.zeros((16,), _f32)


def _seg_compute(out_ref, a_ref, b_ref):
    @plsc.parallel_loop(0, ROWS * 8, unroll=2)
    def _(i):
        j = i // 8
        k = (i % 8) * 16
        out_ref[j, pl.ds(k, 16)] = a_ref[j, pl.ds(k, 16)] * RP + b_ref[j, pl.ds(k, 16)]


def _scale_rows(mbuf, wbuf):
    @plsc.parallel_loop(0, ECH // 16, unroll=4)
    def _(g):
        wv = wbuf[g // 8, pl.ds((g % 8) * 16, 16)]
        for j in range(16):
            e = g * 16 + j
            mbuf[e, :] = mbuf[e, :] * wv[j]


def _sc_weights_body(e3_ref, w_ref,
                     cnt_sh, lin0, d0, d1, e0, e1, g0, g1,
                     wb0, wb1, ones,
                     sem_st0, sem_st1, sem_sc0, sem_sc1, sem_io):
    c = lax.axis_index("c")
    s = lax.axis_index("s")
    wid = s * NCORE + c
    dbuf = (d0, d1)
    ebuf = (e0, e1)
    gbuf = (g0, g1)
    wbuf = (wb0, wb1)
    sem_st = (sem_st0, sem_st1)
    sem_sc = (sem_sc0, sem_sc1)

    _zero_vec16(lin0, BIN_CH // 16)
    zdesc = []
    for k in range(BIN_T // BIN_CH):
        off = pl.multiple_of(s * BIN_T + k * BIN_CH, 16)
        zdesc.append(pltpu.async_copy(lin0, cnt_sh.at[pl.ds(off, BIN_CH)], sem_io))

    @pl.loop(0, ROWS * 8)
    def _(i):
        ones[i // 8, pl.ds((i % 8) * 16, 16)] = jnp.ones((16,), _f32)

    for d in zdesc:
        d.wait()
    plsc.subcore_barrier()

    st_descs = {}
    sc_descs = {}

    def issue_cnt_stage(ci):
        p = ci & 1
        row = pl.multiple_of(s * NCH_C * ROWS + ci * ROWS, 8)
        st_descs[ci] = [
            pltpu.async_copy(e3_ref.at[1, pl.ds(row, ROWS), :], dbuf[p], sem_st[p]),
            pltpu.async_copy(e3_ref.at[2, pl.ds(row, ROWS), :], ebuf[p], sem_st[p]),
        ]

    issue_cnt_stage(0)
    for ci in range(NCH_C):
        p = ci & 1
        for d in st_descs.pop(ci):
            d.wait()
        if ci + 1 < NCH_C:
            if ci >= 1:
                for d in sc_descs.pop(ci - 1):
                    d.wait()
            issue_cnt_stage(ci + 1)
        _seg_compute(gbuf[p], dbuf[p], ebuf[p])
        sc_descs[ci] = [
            pltpu.async_copy(ones.at[j], cnt_sh.at[gbuf[p].at[j]], sem_sc[p], add=True)
            for j in range(ROWS)
        ]
    for ci in list(sc_descs):
        for d in sc_descs.pop(ci):
            d.wait()
    plsc.subcore_barrier()

    wout_descs = {}
    for ci in range(NCH_E):
        p = ci & 1
        base_e = pl.multiple_of(wid * (EP // (NSUB * NCORE)) + ci * ECH, ECH)
        row = pl.multiple_of(base_e // 128, 8)
        a = pltpu.async_copy(e3_ref.at[1, pl.ds(row, ROWS), :], dbuf[p], sem_st[p])
        b = pltpu.async_copy(e3_ref.at[2, pl.ds(row, ROWS), :], ebuf[p], sem_st[p])
        a.wait()
        b.wait()
        _seg_compute(gbuf[p], dbuf[p], ebuf[p])
        if ci >= 2:
            wout_descs.pop(ci - 2).wait()
        gd = [pltpu.async_copy(cnt_sh.at[gbuf[p].at[j]], wbuf[p].at[j], sem_sc[p])
              for j in range(ROWS)]
        for d in gd:
            d.wait()

        @pl.loop(0, ECH // 16)
        def _(i):
            v = wbuf[p][i // 8, pl.ds((i % 8) * 16, 16)]
            wbuf[p][i // 8, pl.ds((i % 8) * 16, 16)] = 1.0 / jnp.maximum(v, 1.0)

        wout_descs[ci] = pltpu.async_copy(wbuf[p], w_ref.at[pl.ds(row, ROWS), :], sem_io)
    for ci in list(wout_descs):
        wout_descs.pop(ci).wait()


def _sc_weights(edges3):
    f = pl.kernel(
        _sc_weights_body,
        out_type=jax.ShapeDtypeStruct((EP // 128, 128), _f32),
        mesh=plsc.VectorSubcoreMesh(core_axis_name="c", subcore_axis_name="s"),
        compiler_params=pltpu.CompilerParams(use_tc_tiling_on_sc=False),
        scratch_types=[
            pltpu.VMEM_SHARED((NRP,), _f32),
            pltpu.VMEM((BIN_CH,), _f32),
            pltpu.VMEM((ROWS, 128), _i32),
            pltpu.VMEM((ROWS, 128), _i32),
            pltpu.VMEM((ROWS, 128), _i32),
            pltpu.VMEM((ROWS, 128), _i32),
            pltpu.VMEM((ROWS, 128), _i32),
            pltpu.VMEM((ROWS, 128), _i32),
            pltpu.VMEM((ROWS, 128), _f32),
            pltpu.VMEM((ROWS, 128), _f32),
            pltpu.VMEM((ROWS, 128), _f32),
            pltpu.SemaphoreType.DMA,
            pltpu.SemaphoreType.DMA,
            pltpu.SemaphoreType.DMA,
            pltpu.SemaphoreType.DMA,
            pltpu.SemaphoreType.DMA,
        ],
    )
    return f(edges3)


def _sc_layer_body(y_ref, e3_ref, w_in_ref,
                   part0_ref, part1_ref,
                   acc_sh, stage, s0, s1, d0, d1, e0, e1,
                   wb0, wb1, m0, m1,
                   sem_st0, sem_st1, sem_g0, sem_g1, sem_sc0, sem_sc1):
    c = lax.axis_index("c")
    s = lax.axis_index("s")
    wid = s * NCORE + c
    sbuf = (s0, s1)
    dbuf = (d0, d1)
    ebuf = (e0, e1)
    wbuf = (wb0, wb1)
    mbuf = (m0, m1)
    sem_st = (sem_st0, sem_st1)
    sem_g = (sem_g0, sem_g1)
    sem_sc = (sem_sc0, sem_sc1)

    @pl.loop(0, ACC_T)
    def _(i):
        stage[i, :] = jnp.zeros((16,), _f32)
    acc_off = pl.multiple_of(s * ACC_T, 8)
    pltpu.sync_copy(stage, acc_sh.at[pl.ds(acc_off, ACC_T), :])
    plsc.subcore_barrier()

    st_descs = {}
    sc_descs = {}

    def issue_stage(ci):
        p = ci & 1
        base_e = pl.multiple_of(wid * (EP // (NSUB * NCORE)) + ci * ECH, ECH)
        row = pl.multiple_of(base_e // 128, 8)
        st_descs[ci] = [
            pltpu.async_copy(e3_ref.at[0, pl.ds(row, ROWS), :], sbuf[p], sem_st[p]),
            pltpu.async_copy(e3_ref.at[1, pl.ds(row, ROWS), :], dbuf[p], sem_st[p]),
            pltpu.async_copy(e3_ref.at[2, pl.ds(row, ROWS), :], ebuf[p], sem_st[p]),
            pltpu.async_copy(w_in_ref.at[pl.ds(row, ROWS), :], wbuf[p], sem_st[p]),
        ]

    def fire_gathers(ci):
        p = ci & 1
        return [pltpu.async_copy(y_ref.at[sbuf[p].at[j]],
                                 mbuf[p].at[pl.ds(j * 128, 128), :], sem_g[p])
                for j in range(ROWS)]

    issue_stage(0)
    for d in st_descs.pop(0):
        d.wait()
    _seg_compute(sbuf[0], sbuf[0], ebuf[0])
    g_descs = {0: fire_gathers(0)}
    for ci in range(NCH_E):
        p = ci & 1
        if ci + 1 < NCH_E:
            if ci >= 1:
                for d in sc_descs.pop(ci - 1):
                    d.wait()
            issue_stage(ci + 1)
            for d in st_descs.pop(ci + 1):
                d.wait()
            _seg_compute(sbuf[1 - p], sbuf[1 - p], ebuf[1 - p])
            g_descs[ci + 1] = fire_gathers(ci + 1)
        for d in g_descs.pop(ci):
            d.wait()
        _scale_rows(mbuf[p], wbuf[p])
        sc_descs[ci] = [
            pltpu.async_copy(mbuf[p].at[pl.ds(j * 128, 128), :],
                             acc_sh.at[dbuf[p].at[j]], sem_sc[p], add=True)
            for j in range(ROWS)
        ]
    for ci in list(sc_descs):
        for d in sc_descs.pop(ci):
            d.wait()
    plsc.subcore_barrier()

    pltpu.sync_copy(acc_sh.at[pl.ds(acc_off, ACC_T), :], stage)

    @pl.when(c == 0)
    def _():
        pltpu.sync_copy(stage, part0_ref.at[pl.ds(acc_off, ACC_T), pl.ds(0, HID)])

    @pl.when(c == 1)
    def _():
        pltpu.sync_copy(stage, part1_ref.at[pl.ds(acc_off, ACC_T), pl.ds(0, HID)])


def _sc_layer(y, edges3, w_e):
    f = pl.kernel(
        _sc_layer_body,
        out_type=[
            jax.ShapeDtypeStruct((NP, 128), _f32),
            jax.ShapeDtypeStruct((NP, 128), _f32),
        ],
        mesh=plsc.VectorSubcoreMesh(core_axis_name="c", subcore_axis_name="s"),
        compiler_params=pltpu.CompilerParams(use_tc_tiling_on_sc=False),
        scratch_types=[
            pltpu.VMEM_SHARED((NP, HID), _f32),
            pltpu.VMEM((ACC_T, HID), _f32),
            pltpu.VMEM((ROWS, 128), _i32),
            pltpu.VMEM((ROWS, 128), _i32),
            pltpu.VMEM((ROWS, 128), _i32),
            pltpu.VMEM((ROWS, 128), _i32),
            pltpu.VMEM((ROWS, 128), _i32),
            pltpu.VMEM((ROWS, 128), _i32),
            pltpu.VMEM((ROWS, 128), _f32),
            pltpu.VMEM((ROWS, 128), _f32),
            pltpu.VMEM((ECH, HID), _f32),
            pltpu.VMEM((ECH, HID), _f32),
            pltpu.SemaphoreType.DMA,
            pltpu.SemaphoreType.DMA,
            pltpu.SemaphoreType.DMA,
            pltpu.SemaphoreType.DMA,
            pltpu.SemaphoreType.DMA,
            pltpu.SemaphoreType.DMA,
        ],
    )
    return f(y, edges3, w_e)


def kernel(x, edge_index, edge_type, W1, root1, b1, W2, root2, b2):
    pad = EP - E
    i = jnp.arange(pad, dtype=_i32)
    base3 = jnp.concatenate([edge_index.astype(_i32),
                             edge_type.astype(_i32)[None]], axis=0)
    pad3 = jnp.stack([i % N, N + (i % RP), i % R])
    edges3 = jnp.concatenate([base3.reshape(3, E // 128, 128),
                              pad3.reshape(3, (EP - E) // 128, 128)], axis=1)

    w1c = jnp.pad(jnp.transpose(W1, (1, 0, 2)).reshape(IN_CH, R * HID),
                  ((0, 0), (0, (RP - R) * HID)))
    w2c = jnp.pad(jnp.transpose(W2, (1, 0, 2)).reshape(HID, R * HID),
                  ((0, 0), (0, (RP - R) * HID)))

    w_e = _sc_weights(edges3)
    y1, xr1 = _tc1(x, w1c, root1, b1.reshape(1, HID))
    p0, p1 = _sc_layer(y1.reshape(NRY, HID), edges3, w_e)
    y2, xr2 = _tc2(p0, p1, xr1, w2c, root2, b2.reshape(1, HID))
    q0, q1 = _sc_layer(y2.reshape(NRY, HID), edges3, w_e)
    return _tc3(q0, q1, xr2)


# speedup vs baseline: 1.0263x; 1.0263x over previous
import jax
import jax.numpy as jnp
from jax import lax
from jax.experimental import pallas as pl
from jax.experimental.pallas import tpu as pltpu
from jax.experimental.pallas import tpu_sc as plsc

N = 10000
E = 320000
R = 45
IN_CH = 128
HID = 16
RP = 48
NP = 10112
NRP = NP * RP
NRY = N * RP
EP = 327680
ECH = 2048
ROWS = ECH // 128
NSUB = 16
NCORE = 2
BIN_T = NRP // NSUB
BIN_CH = 5056
ACC_T = NP // NSUB
NCH_E = EP // ECH // (NSUB * NCORE)
NCH_C = EP // ECH // NSUB

_f32 = jnp.float32
_i32 = jnp.int32


def _tc1_body(x_ref, w_ref, r_ref, b_ref, y_ref, xr_ref):
    xb = x_ref[...]
    y = jnp.dot(xb, w_ref[...], preferred_element_type=_f32)
    y_ref[...] = y.reshape(y_ref.shape)
    xr_ref[...] = jnp.dot(xb, r_ref[...], preferred_element_type=_f32) + b_ref[...]


def _tc2_body(p0_ref, p1_ref, xr_ref, w_ref, r_ref, b_ref, y_ref, xr2_ref):
    h = jnp.maximum(p0_ref[:, :HID] + p1_ref[:, :HID] + xr_ref[...], 0.0)
    y = jnp.dot(h, w_ref[...], preferred_element_type=_f32)
    y_ref[...] = y.reshape(y_ref.shape)
    xr2_ref[...] = jnp.dot(h, r_ref[...], preferred_element_type=_f32) + b_ref[...]


def _tc3_body(p0_ref, p1_ref, xr_ref, o_ref):
    o_ref[...] = p0_ref[:, :HID] + p1_ref[:, :HID] + xr_ref[...]


def _tc1(x, w1c, root1, b1):
    blk = 2000
    return pl.pallas_call(
        _tc1_body,
        grid=(N // blk,),
        in_specs=[
            pl.BlockSpec((blk, IN_CH), lambda i: (i, 0)),
            pl.BlockSpec((IN_CH, RP * HID), lambda i: (0, 0)),
            pl.BlockSpec((IN_CH, HID), lambda i: (0, 0)),
            pl.BlockSpec((1, HID), lambda i: (0, 0)),
        ],
        out_specs=[
            pl.BlockSpec((blk * 6, 128), lambda i: (i, 0)),
            pl.BlockSpec((blk, HID), lambda i: (i, 0)),
        ],
        out_shape=[
            jax.ShapeDtypeStruct((N * 6, 128), _f32),
            jax.ShapeDtypeStruct((N, HID), _f32),
        ],
    )(x, w1c, root1, b1)


def _tc2(p0, p1, xr1, w2c, root2, b2):
    blk = 2000
    return pl.pallas_call(
        _tc2_body,
        grid=(N // blk,),
        in_specs=[
            pl.BlockSpec((blk, 128), lambda i: (i, 0)),
            pl.BlockSpec((blk, 128), lambda i: (i, 0)),
            pl.BlockSpec((blk, HID), lambda i: (i, 0)),
            pl.BlockSpec((HID, RP * HID), lambda i: (0, 0)),
            pl.BlockSpec((HID, HID), lambda i: (0, 0)),
            pl.BlockSpec((1, HID), lambda i: (0, 0)),
        ],
        out_specs=[
            pl.BlockSpec((blk * 6, 128), lambda i: (i, 0)),
            pl.BlockSpec((blk, HID), lambda i: (i, 0)),
        ],
        out_shape=[
            jax.ShapeDtypeStruct((N * 6, 128), _f32),
            jax.ShapeDtypeStruct((N, HID), _f32),
        ],
    )(p0, p1, xr1, w2c, root2, b2)


def _tc3(p0, p1, xr2):
    blk = 2000
    return pl.pallas_call(
        _tc3_body,
        grid=(N // blk,),
        in_specs=[
            pl.BlockSpec((blk, 128), lambda i: (i, 0)),
            pl.BlockSpec((blk, 128), lambda i: (i, 0)),
            pl.BlockSpec((blk, HID), lambda i: (i, 0)),
        ],
        out_specs=pl.BlockSpec((blk, HID), lambda i: (i, 0)),
        out_shape=jax.ShapeDtypeStruct((N, HID), _f32),
    )(p0, p1, xr2)


def _zero_vec16(ref, nvec):
    @pl.loop(0, nvec)
    def _(i):
        ref[pl.ds(i * 16, 16)] = jnp.zeros((16,), _f32)


def _seg_compute(out_ref, a_ref, b_ref):
    @plsc.parallel_loop(0, ROWS * 8, unroll=2)
    def _(i):
        j = i // 8
        k = (i % 8) * 16
        out_ref[j, pl.ds(k, 16)] = a_ref[j, pl.ds(k, 16)] * RP + b_ref[j, pl.ds(k, 16)]


def _scale_rows(mbuf, wbuf):
    @plsc.parallel_loop(0, ECH // 16, unroll=4)
    def _(g):
        wv = wbuf[pl.ds(g * 16, 16)]
        for j in range(16):
            e = g * 16 + j
            mbuf[e, :] = mbuf[e, :] * wv[j]


def _sc_weights_body(e3_ref, w_ref,
                     cnt_sh, lin0, d0, d1, e0, e1, g0, g1,
                     wb0, wb1, ones,
                     sem_st0, sem_st1, sem_sc0, sem_sc1, sem_io):
    c = lax.axis_index("c")
    s = lax.axis_index("s")
    wid = s * NCORE + c
    dbuf = (d0, d1)
    ebuf = (e0, e1)
    gbuf = (g0, g1)
    wbuf = (wb0, wb1)
    sem_st = (sem_st0, sem_st1)
    sem_sc = (sem_sc0, sem_sc1)

    _zero_vec16(lin0, BIN_CH // 16)
    zdesc = []
    for k in range(BIN_T // BIN_CH):
        off = pl.multiple_of(s * BIN_T + k * BIN_CH, 16)
        zdesc.append(pltpu.async_copy(lin0, cnt_sh.at[pl.ds(off, BIN_CH)], sem_io))

    @pl.loop(0, 8)
    def _(i):
        ones[pl.ds(i * 16, 16)] = jnp.ones((16,), _f32)

    for d in zdesc:
        d.wait()
    plsc.subcore_barrier()

    st_descs = {}
    sc_descs = {}

    def issue_cnt_stage(ci):
        p = ci & 1
        row = pl.multiple_of(s * NCH_C * ROWS + ci * ROWS, 8)
        st_descs[ci] = [
            pltpu.async_copy(e3_ref.at[1, pl.ds(row, ROWS), :], dbuf[p], sem_st[p]),
            pltpu.async_copy(e3_ref.at[2, pl.ds(row, ROWS), :], ebuf[p], sem_st[p]),
        ]

    issue_cnt_stage(0)
    for ci in range(NCH_C):
        p = ci & 1
        for d in st_descs.pop(ci):
            d.wait()
        if ci + 1 < NCH_C:
            if ci >= 1:
                for d in sc_descs.pop(ci - 1):
                    d.wait()
            issue_cnt_stage(ci + 1)
        _seg_compute(gbuf[p], dbuf[p], ebuf[p])
        sc_descs[ci] = [
            pltpu.async_copy(ones, cnt_sh.at[gbuf[p].at[j]], sem_sc[p], add=True)
            for j in range(ROWS)
        ]
    for ci in list(sc_descs):
        for d in sc_descs.pop(ci):
            d.wait()
    plsc.subcore_barrier()

    wout_descs = {}
    for ci in range(NCH_E):
        p = ci & 1
        base_e = pl.multiple_of(wid * (EP // (NSUB * NCORE)) + ci * ECH, ECH)
        row = pl.multiple_of(base_e // 128, 8)
        a = pltpu.async_copy(e3_ref.at[1, pl.ds(row, ROWS), :], dbuf[p], sem_st[p])
        b = pltpu.async_copy(e3_ref.at[2, pl.ds(row, ROWS), :], ebuf[p], sem_st[p])
        a.wait()
        b.wait()
        _seg_compute(gbuf[p], dbuf[p], ebuf[p])
        if ci >= 2:
            wout_descs.pop(ci - 2).wait()
        gd = [pltpu.async_copy(cnt_sh.at[gbuf[p].at[j]],
                               wbuf[p].at[pl.ds(j * 128, 128)], sem_sc[p])
              for j in range(ROWS)]
        for d in gd:
            d.wait()

        @pl.loop(0, ECH // 16)
        def _(i):
            v = wbuf[p][pl.ds(i * 16, 16)]
            wbuf[p][pl.ds(i * 16, 16)] = 1.0 / jnp.maximum(v, 1.0)

        wout_descs[ci] = pltpu.async_copy(wbuf[p], w_ref.at[pl.ds(base_e, ECH)], sem_io)
    for ci in list(wout_descs):
        wout_descs.pop(ci).wait()


def _sc_weights(edges3):
    f = pl.kernel(
        _sc_weights_body,
        out_type=jax.ShapeDtypeStruct((EP,), _f32),
        mesh=plsc.VectorSubcoreMesh(core_axis_name="c", subcore_axis_name="s"),
        compiler_params=pltpu.CompilerParams(use_tc_tiling_on_sc=False),
        scratch_types=[
            pltpu.VMEM_SHARED((NRP,), _f32),
            pltpu.VMEM((BIN_CH,), _f32),
            pltpu.VMEM((ROWS, 128), _i32),
            pltpu.VMEM((ROWS, 128), _i32),
            pltpu.VMEM((ROWS, 128), _i32),
            pltpu.VMEM((ROWS, 128), _i32),
            pltpu.VMEM((ROWS, 128), _i32),
            pltpu.VMEM((ROWS, 128), _i32),
            pltpu.VMEM((ECH,), _f32),
            pltpu.VMEM((ECH,), _f32),
            pltpu.VMEM((128,), _f32),
            pltpu.SemaphoreType.DMA,
            pltpu.SemaphoreType.DMA,
            pltpu.SemaphoreType.DMA,
            pltpu.SemaphoreType.DMA,
            pltpu.SemaphoreType.DMA,
        ],
    )
    return f(edges3)


def _sc_layer_body(y_ref, e3_ref, w_in_ref,
                   part0_ref, part1_ref,
                   acc_sh, stage, s0, s1, d0, d1, e0, e1,
                   wb0, wb1, m0, m1,
                   sem_st0, sem_st1, sem_g0, sem_g1, sem_sc0, sem_sc1):
    c = lax.axis_index("c")
    s = lax.axis_index("s")
    wid = s * NCORE + c
    sbuf = (s0, s1)
    dbuf = (d0, d1)
    ebuf = (e0, e1)
    wbuf = (wb0, wb1)
    mbuf = (m0, m1)
    sem_st = (sem_st0, sem_st1)
    sem_g = (sem_g0, sem_g1)
    sem_sc = (sem_sc0, sem_sc1)

    @pl.loop(0, ACC_T)
    def _(i):
        stage[i, :] = jnp.zeros((16,), _f32)
    acc_off = pl.multiple_of(s * ACC_T, 8)
    pltpu.sync_copy(stage, acc_sh.at[pl.ds(acc_off, ACC_T), :])
    plsc.subcore_barrier()

    st_descs = {}
    sc_descs = {}

    def issue_stage(ci):
        p = ci & 1
        base_e = pl.multiple_of(wid * (EP // (NSUB * NCORE)) + ci * ECH, ECH)
        row = pl.multiple_of(base_e // 128, 8)
        st_descs[ci] = [
            pltpu.async_copy(e3_ref.at[0, pl.ds(row, ROWS), :], sbuf[p], sem_st[p]),
            pltpu.async_copy(e3_ref.at[1, pl.ds(row, ROWS), :], dbuf[p], sem_st[p]),
            pltpu.async_copy(e3_ref.at[2, pl.ds(row, ROWS), :], ebuf[p], sem_st[p]),
            pltpu.async_copy(w_in_ref.at[pl.ds(base_e, ECH)], wbuf[p], sem_st[p]),
        ]

    def fire_gathers(ci):
        p = ci & 1
        return [pltpu.async_copy(y_ref.at[sbuf[p].at[j]],
                                 mbuf[p].at[pl.ds(j * 128, 128), :], sem_g[p])
                for j in range(ROWS)]

    issue_stage(0)
    for d in st_descs.pop(0):
        d.wait()
    _seg_compute(sbuf[0], sbuf[0], ebuf[0])
    g_descs = {0: fire_gathers(0)}
    for ci in range(NCH_E):
        p = ci & 1
        if ci + 1 < NCH_E:
            if ci >= 1:
                for d in sc_descs.pop(ci - 1):
                    d.wait()
            issue_stage(ci + 1)
            for d in st_descs.pop(ci + 1):
                d.wait()
            _seg_compute(sbuf[1 - p], sbuf[1 - p], ebuf[1 - p])
            g_descs[ci + 1] = fire_gathers(ci + 1)
        for d in g_descs.pop(ci):
            d.wait()
        _scale_rows(mbuf[p], wbuf[p])
        sc_descs[ci] = [
            pltpu.async_copy(mbuf[p].at[pl.ds(j * 128, 128), :],
                             acc_sh.at[dbuf[p].at[j]], sem_sc[p], add=True)
            for j in range(ROWS)
        ]
    for ci in list(sc_descs):
        for d in sc_descs.pop(ci):
            d.wait()
    plsc.subcore_barrier()

    pltpu.sync_copy(acc_sh.at[pl.ds(acc_off, ACC_T), :], stage)

    @pl.when(c == 0)
    def _():
        pltpu.sync_copy(stage, part0_ref.at[pl.ds(acc_off, ACC_T), pl.ds(0, HID)])

    @pl.when(c == 1)
    def _():
        pltpu.sync_copy(stage, part1_ref.at[pl.ds(acc_off, ACC_T), pl.ds(0, HID)])


def _sc_layer(y, edges3, w_e):
    f = pl.kernel(
        _sc_layer_body,
        out_type=[
            jax.ShapeDtypeStruct((NP, 128), _f32),
            jax.ShapeDtypeStruct((NP, 128), _f32),
        ],
        mesh=plsc.VectorSubcoreMesh(core_axis_name="c", subcore_axis_name="s"),
        compiler_params=pltpu.CompilerParams(use_tc_tiling_on_sc=False),
        scratch_types=[
            pltpu.VMEM_SHARED((NP, HID), _f32),
            pltpu.VMEM((ACC_T, HID), _f32),
            pltpu.VMEM((ROWS, 128), _i32),
            pltpu.VMEM((ROWS, 128), _i32),
            pltpu.VMEM((ROWS, 128), _i32),
            pltpu.VMEM((ROWS, 128), _i32),
            pltpu.VMEM((ROWS, 128), _i32),
            pltpu.VMEM((ROWS, 128), _i32),
            pltpu.VMEM((ECH,), _f32),
            pltpu.VMEM((ECH,), _f32),
            pltpu.VMEM((ECH, HID), _f32),
            pltpu.VMEM((ECH, HID), _f32),
            pltpu.SemaphoreType.DMA,
            pltpu.SemaphoreType.DMA,
            pltpu.SemaphoreType.DMA,
            pltpu.SemaphoreType.DMA,
            pltpu.SemaphoreType.DMA,
            pltpu.SemaphoreType.DMA,
        ],
    )
    return f(y, edges3, w_e)


def kernel(x, edge_index, edge_type, W1, root1, b1, W2, root2, b2):
    pad = EP - E
    i = jnp.arange(pad, dtype=_i32)
    base3 = jnp.concatenate([edge_index.astype(_i32),
                             edge_type.astype(_i32)[None]], axis=0)
    pad3 = jnp.stack([i % N, N + (i % RP), i % R])
    edges3 = jnp.concatenate([base3.reshape(3, E // 128, 128),
                              pad3.reshape(3, (EP - E) // 128, 128)], axis=1)

    w1c = jnp.pad(jnp.transpose(W1, (1, 0, 2)).reshape(IN_CH, R * HID),
                  ((0, 0), (0, (RP - R) * HID)))
    w2c = jnp.pad(jnp.transpose(W2, (1, 0, 2)).reshape(HID, R * HID),
                  ((0, 0), (0, (RP - R) * HID)))

    w_e = _sc_weights(edges3)
    y1, xr1 = _tc1(x, w1c, root1, b1.reshape(1, HID))
    p0, p1 = _sc_layer(y1.reshape(NRY, HID), edges3, w_e)
    y2, xr2 = _tc2(p0, p1, xr1, w2c, root2, b2.reshape(1, HID))
    q0, q1 = _sc_layer(y2.reshape(NRY, HID), edges3, w_e)
    return _tc3(q0, q1, xr2)
